# baseline (speedup 1.0000x reference)
import functools

import jax
import jax.numpy as jnp
from jax import lax
from jax.experimental import pallas as pl
from jax.experimental.pallas import tpu as pltpu
from jax.experimental.pallas import tpu_sc as plsc

_MAX_ATTENTION_SIZE = 4096
_SINK = 64


def _ring_copy(jobs, bufs, gsems, ssems, nbuf):
    nj = len(jobs)
    for j in range(min(nbuf - 1, nj)):
        jobs[j][0](bufs[j], gsems[j]).start()
    for j in range(nj):
        cur = j % nbuf
        if j + nbuf - 1 < nj:
            pf = (j + nbuf - 1) % nbuf
            if j >= 1:
                jobs[j - 1][1](bufs[pf], ssems[pf]).wait()
            jobs[j + nbuf - 1][0](bufs[pf], gsems[pf]).start()
        jobs[j][0](bufs[cur], gsems[cur]).wait()
        jobs[j][1](bufs[cur], ssems[cur]).start()
    for j in range(max(0, nj - nbuf), nj):
        jobs[j][1](bufs[j % nbuf], ssems[j % nbuf]).wait()


def kernel(cache_k, cache_v, k, v, global_end_index, local_end_index, num_new_tokens):
    BS, S, H, D = cache_k.shape
    NN = k.shape[1]
    NR = S - NN - _SINK
    QUARTER = NR // 4

    lei = jnp.asarray(local_end_index, jnp.int32)
    nnt = jnp.asarray(num_new_tokens, jnp.int32)
    num_evicted = lei + nnt - S
    src0 = jnp.clip(_SINK + num_evicted, 0, S - NR).astype(jnp.int32)
    new_local_end = (lei + nnt - num_evicted).astype(jnp.int32)
    window_start = jnp.maximum(new_local_end - _MAX_ATTENTION_SIZE, 0).astype(jnp.int32)

    src0_v = jnp.full((16,), src0, jnp.int32)

    C = 40
    NBUF = 2
    NCH = QUARTER // C
    assert QUARTER % C == 0

    mesh = plsc.VectorSubcoreMesh(core_axis_name="c", subcore_axis_name="s")

    @functools.partial(
        pl.kernel,
        out_type=jax.ShapeDtypeStruct((BS, S, H, D), jnp.float32),
        mesh=mesh,
        scratch_types=[
            pltpu.VMEM((16,), jnp.int32),
        ] + [pltpu.SemaphoreType.DMA] * (2 * NBUF),
    )
    def _sc_copy(cv_h, vn_h, s0_h, ov_h, s0_vm, *sems):
        c = lax.axis_index("c")
        s = lax.axis_index("s")
        wid = s * 2 + c
        b = wid // 4
        q = wid % 4
        pltpu.sync_copy(s0_h, s0_vm)
        s0 = s0_vm[...][0]
        src_base = s0 + q * QUARTER
        dst_base = _SINK + q * QUARTER
        gsems = sems[:NBUF]
        ssems = sems[NBUF:]

        def scoped(*bufs):
            def gather(i):
                def mk(buf, sem):
                    return pltpu.make_async_copy(
                        cv_h.at[b, pl.ds(src_base + i * C, C)], buf, sem)
                return mk

            def scatter(i):
                def mk(buf, sem):
                    return pltpu.make_async_copy(
                        buf, ov_h.at[b, pl.ds(dst_base + i * C, C)], sem)
                return mk

            _ring_copy([(gather(i), scatter(i)) for i in range(NCH)],
                       bufs, gsems, ssems, NBUF)

            def tail(tsrc_h, src_row, dst_row, nrows):
                off = 0
                while off < nrows:
                    m = min(C, nrows - off)
                    pltpu.sync_copy(tsrc_h.at[b, pl.ds(src_row + off, m)],
                                    bufs[0].at[pl.ds(0, m)])
                    pltpu.sync_copy(bufs[0].at[pl.ds(0, m)],
                                    ov_h.at[b, pl.ds(dst_row + off, m)])
                    off += m

            @pl.when(q == 0)
            def _():
                tail(cv_h, 0, 0, _SINK)

            @pl.when(q == 3)
            def _():
                tail(vn_h, 0, S - NN, NN)

        pl.run_scoped(scoped, *[pltpu.VMEM((C, H, D), jnp.float32)
                                for _ in range(NBUF)])

    CT = 1000
    NBUFT = 4
    NCHT = NR // CT
    assert NR % CT == 0

    def _tc_body(s0_ref, ck_h, kn_h, ok_h, *sems):
        s0 = s0_ref[0]

        copies = []
        for b in range(BS):
            copies.append(lambda sem, b=b: pltpu.make_async_copy(
                ck_h.at[b, pl.ds(0, _SINK)], ok_h.at[b, pl.ds(0, _SINK)], sem))
            for i in range(NCHT):
                copies.append(lambda sem, b=b, i=i: pltpu.make_async_copy(
                    ck_h.at[b, pl.ds(s0 + i * CT, CT)],
                    ok_h.at[b, pl.ds(_SINK + i * CT, CT)], sem))
            copies.append(lambda sem, b=b: pltpu.make_async_copy(
                kn_h.at[b], ok_h.at[b, pl.ds(S - NN, NN)], sem))

        nsem = len(sems)
        for j, mk in enumerate(copies):
            mk(sems[j % nsem]).start()
        for j, mk in enumerate(copies):
            mk(sems[j % nsem]).wait()

    ov = _sc_copy(cache_v, v, src0_v)

    ok = pl.pallas_call(
        _tc_body,
        out_shape=jax.ShapeDtypeStruct((BS, S, H, D), jnp.float32),
        in_specs=[
            pl.BlockSpec(memory_space=pltpu.SMEM),
            pl.BlockSpec(memory_space=pl.ANY),
            pl.BlockSpec(memory_space=pl.ANY),
        ],
        out_specs=pl.BlockSpec(memory_space=pl.ANY),
        scratch_shapes=[pltpu.SemaphoreType.DMA] * 8,
    )(jnp.full((1,), src0, jnp.int32), cache_k, k)

    return (ok, ov, window_start, new_local_end)


# speedup vs baseline: 6.9387x; 6.9387x over previous
import functools

import jax
import jax.numpy as jnp
from jax import lax
from jax.experimental import pallas as pl
from jax.experimental.pallas import tpu as pltpu
from jax.experimental.pallas import tpu_sc as plsc

_MAX_ATTENTION_SIZE = 4096
_SINK = 64


def _ring_copy(jobs, bufs, gsems, ssems, nbuf):
    nj = len(jobs)
    for j in range(min(nbuf - 1, nj)):
        jobs[j][0](bufs[j], gsems[j]).start()
    for j in range(nj):
        cur = j % nbuf
        if j + nbuf - 1 < nj:
            pf = (j + nbuf - 1) % nbuf
            if j >= 1:
                jobs[j - 1][1](bufs[pf], ssems[pf]).wait()
            jobs[j + nbuf - 1][0](bufs[pf], gsems[pf]).start()
        jobs[j][0](bufs[cur], gsems[cur]).wait()
        jobs[j][1](bufs[cur], ssems[cur]).start()
    for j in range(max(0, nj - nbuf), nj):
        jobs[j][1](bufs[j % nbuf], ssems[j % nbuf]).wait()


def kernel(cache_k, cache_v, k, v, global_end_index, local_end_index, num_new_tokens):
    BS, S, H, D = cache_k.shape
    NN = k.shape[1]
    NR = S - NN - _SINK
    QUARTER = NR // 4

    lei = jnp.asarray(local_end_index, jnp.int32)
    nnt = jnp.asarray(num_new_tokens, jnp.int32)
    num_evicted = lei + nnt - S
    src0 = jnp.clip(_SINK + num_evicted, 0, S - NR).astype(jnp.int32)
    new_local_end = (lei + nnt - num_evicted).astype(jnp.int32)
    window_start = jnp.maximum(new_local_end - _MAX_ATTENTION_SIZE, 0).astype(jnp.int32)

    src0_v = jnp.full((16,), src0, jnp.int32)

    C = 40
    NBUF = 2
    NCH = QUARTER // C
    assert QUARTER % C == 0

    mesh = plsc.VectorSubcoreMesh(core_axis_name="c", subcore_axis_name="s")

    @functools.partial(
        pl.kernel,
        out_type=jax.ShapeDtypeStruct((BS, S, H, D), jnp.float32),
        mesh=mesh,
        scratch_types=[
            pltpu.VMEM((16,), jnp.int32),
        ] + [pltpu.SemaphoreType.DMA] * (2 * NBUF),
    )
    def _sc_copy(cv_h, vn_h, s0_h, ov_h, s0_vm, *sems):
        c = lax.axis_index("c")
        s = lax.axis_index("s")
        wid = s * 2 + c
        b = wid // 4
        q = wid % 4
        pltpu.sync_copy(s0_h, s0_vm)
        s0 = s0_vm[...][0]
        src_base = s0 + q * QUARTER
        dst_base = _SINK + q * QUARTER
        gsems = sems[:NBUF]
        ssems = sems[NBUF:]

        def scoped(*bufs):
            def gather(i):
                def mk(buf, sem):
                    return pltpu.make_async_copy(
                        cv_h.at[b, pl.ds(src_base + i * C, C)], buf, sem)
                return mk

            def scatter(i):
                def mk(buf, sem):
                    return pltpu.make_async_copy(
                        buf, ov_h.at[b, pl.ds(dst_base + i * C, C)], sem)
                return mk

            _ring_copy([(gather(i), scatter(i)) for i in range(NCH)],
                       bufs, gsems, ssems, NBUF)

            def tail(tsrc_h, src_row, dst_row, nrows):
                off = 0
                while off < nrows:
                    m = min(C, nrows - off)
                    pltpu.sync_copy(tsrc_h.at[b, pl.ds(src_row + off, m)],
                                    bufs[0].at[pl.ds(0, m)])
                    pltpu.sync_copy(bufs[0].at[pl.ds(0, m)],
                                    ov_h.at[b, pl.ds(dst_row + off, m)])
                    off += m

            @pl.when(q == 0)
            def _():
                tail(cv_h, 0, 0, _SINK)

            @pl.when(q == 3)
            def _():
                tail(vn_h, 0, S - NN, NN)

        pl.run_scoped(scoped, *[pltpu.VMEM((C, H, D), jnp.float32)
                                for _ in range(NBUF)])

    CT = 1000
    NBUFT = 4
    NCHT = NR // CT
    assert NR % CT == 0

    def _tc_body(s0_ref, ck_h, kn_h, ok_h, *scratch):
        bufs = scratch[:NBUFT]
        gsems = scratch[NBUFT:2 * NBUFT]
        ssems = scratch[2 * NBUFT:]
        s0 = s0_ref[0]

        jobs = []
        for b in range(BS):
            def mk_pair(b, sref, srow, drow, n, dyn):
                def g(buf, sem):
                    row = (s0 + srow) if dyn else srow
                    return pltpu.make_async_copy(
                        sref.at[b, pl.ds(row, n)], buf.at[pl.ds(0, n)], sem)

                def sc(buf, sem):
                    return pltpu.make_async_copy(
                        buf.at[pl.ds(0, n)], ok_h.at[b, pl.ds(drow, n)], sem)
                return (g, sc)

            jobs.append(mk_pair(b, ck_h, 0, 0, _SINK, False))
            for i in range(NCHT):
                jobs.append(mk_pair(b, ck_h, i * CT, _SINK + i * CT, CT, True))
            jobs.append(mk_pair(b, kn_h, 0, S - NN, NN, False))

        _ring_copy(jobs, bufs, gsems, ssems, NBUFT)

    ov = _sc_copy(cache_v, v, src0_v)

    ok = pl.pallas_call(
        _tc_body,
        out_shape=jax.ShapeDtypeStruct((BS, S, H, D), jnp.float32),
        in_specs=[
            pl.BlockSpec(memory_space=pltpu.SMEM),
            pl.BlockSpec(memory_space=pl.ANY),
            pl.BlockSpec(memory_space=pl.ANY),
        ],
        out_specs=pl.BlockSpec(memory_space=pl.ANY),
        scratch_shapes=[pltpu.VMEM((CT, H, D), jnp.float32)
                        for _ in range(NBUFT)]
        + [pltpu.SemaphoreType.DMA] * (2 * NBUFT),
    )(jnp.full((1,), src0, jnp.int32), cache_k, k)

    return (ok, ov, window_start, new_local_end)


# speedup vs baseline: 7.1965x; 1.0371x over previous
import functools

import jax
import jax.numpy as jnp
from jax import lax
from jax.experimental import pallas as pl
from jax.experimental.pallas import tpu as pltpu
from jax.experimental.pallas import tpu_sc as plsc

_MAX_ATTENTION_SIZE = 4096
_SINK = 64


def _ring_copy(jobs, bufs, gsems, ssems, nbuf):
    nj = len(jobs)
    for j in range(min(nbuf - 1, nj)):
        jobs[j][0](bufs[j], gsems[j]).start()
    for j in range(nj):
        cur = j % nbuf
        if j + nbuf - 1 < nj:
            pf = (j + nbuf - 1) % nbuf
            if j >= 1:
                jobs[j - 1][1](bufs[pf], ssems[pf]).wait()
            jobs[j + nbuf - 1][0](bufs[pf], gsems[pf]).start()
        jobs[j][0](bufs[cur], gsems[cur]).wait()
        jobs[j][1](bufs[cur], ssems[cur]).start()
    for j in range(max(0, nj - nbuf), nj):
        jobs[j][1](bufs[j % nbuf], ssems[j % nbuf]).wait()


def kernel(cache_k, cache_v, k, v, global_end_index, local_end_index, num_new_tokens):
    BS, S, H, D = cache_k.shape
    NN = k.shape[1]
    NR = S - NN - _SINK
    QUARTER = NR // 4

    lei = jnp.asarray(local_end_index, jnp.int32)
    nnt = jnp.asarray(num_new_tokens, jnp.int32)
    num_evicted = lei + nnt - S
    src0 = jnp.clip(_SINK + num_evicted, 0, S - NR).astype(jnp.int32)
    new_local_end = (lei + nnt - num_evicted).astype(jnp.int32)
    window_start = jnp.maximum(new_local_end - _MAX_ATTENTION_SIZE, 0).astype(jnp.int32)

    src0_v = jnp.full((16,), src0, jnp.int32)

    C = 40
    NBUF = 2
    NCH = QUARTER // C
    assert QUARTER % C == 0

    mesh = plsc.VectorSubcoreMesh(core_axis_name="c", subcore_axis_name="s")

    @functools.partial(
        pl.kernel,
        out_type=jax.ShapeDtypeStruct((BS, S, H, D), jnp.float32),
        mesh=mesh,
        scratch_types=[
            pltpu.VMEM((16,), jnp.int32),
        ] + [pltpu.SemaphoreType.DMA] * (2 * NBUF),
    )
    def _sc_copy(cv_h, vn_h, s0_h, ov_h, s0_vm, *sems):
        c = lax.axis_index("c")
        s = lax.axis_index("s")
        wid = s * 2 + c
        b = wid // 4
        q = wid % 4
        pltpu.sync_copy(s0_h, s0_vm)
        s0 = s0_vm[...][0]
        src_base = s0 + q * QUARTER
        dst_base = _SINK + q * QUARTER
        gsems = sems[:NBUF]
        ssems = sems[NBUF:]

        def scoped(*bufs):
            def gather(i):
                def mk(buf, sem):
                    return pltpu.make_async_copy(
                        cv_h.at[b, pl.ds(src_base + i * C, C)], buf, sem)
                return mk

            def scatter(i):
                def mk(buf, sem):
                    return pltpu.make_async_copy(
                        buf, ov_h.at[b, pl.ds(dst_base + i * C, C)], sem)
                return mk

            _ring_copy([(gather(i), scatter(i)) for i in range(NCH)],
                       bufs, gsems, ssems, NBUF)

            def tail(tsrc_h, src_row, dst_row, nrows):
                off = 0
                while off < nrows:
                    m = min(C, nrows - off)
                    pltpu.sync_copy(tsrc_h.at[b, pl.ds(src_row + off, m)],
                                    bufs[0].at[pl.ds(0, m)])
                    pltpu.sync_copy(bufs[0].at[pl.ds(0, m)],
                                    ov_h.at[b, pl.ds(dst_row + off, m)])
                    off += m

            @pl.when(q == 0)
            def _():
                tail(cv_h, 0, 0, _SINK)

            @pl.when(q == 3)
            def _():
                tail(vn_h, 0, S - NN, NN)

        pl.run_scoped(scoped, *[pltpu.VMEM((C, H, D), jnp.float32)
                                for _ in range(NBUF)])

    CT = 1000
    NBUFT = 4
    NCHT = NR // CT
    assert NR % CT == 0

    ok = _sc_copy(cache_k, k, src0_v)
    ov = _sc_copy(cache_v, v, src0_v)

    return (ok, ov, window_start, new_local_end)


# speedup vs baseline: 7.2017x; 1.0007x over previous
import functools

import jax
import jax.numpy as jnp
from jax import lax
from jax.experimental import pallas as pl
from jax.experimental.pallas import tpu as pltpu
from jax.experimental.pallas import tpu_sc as plsc

_MAX_ATTENTION_SIZE = 4096
_SINK = 64


def _ring_copy(jobs, bufs, gsems, ssems, nbuf):
    nj = len(jobs)
    for j in range(min(nbuf - 1, nj)):
        jobs[j][0](bufs[j], gsems[j]).start()
    for j in range(nj):
        cur = j % nbuf
        if j + nbuf - 1 < nj:
            pf = (j + nbuf - 1) % nbuf
            if j >= 1:
                jobs[j - 1][1](bufs[pf], ssems[pf]).wait()
            jobs[j + nbuf - 1][0](bufs[pf], gsems[pf]).start()
        jobs[j][0](bufs[cur], gsems[cur]).wait()
        jobs[j][1](bufs[cur], ssems[cur]).start()
    for j in range(max(0, nj - nbuf), nj):
        jobs[j][1](bufs[j % nbuf], ssems[j % nbuf]).wait()


def kernel(cache_k, cache_v, k, v, global_end_index, local_end_index, num_new_tokens):
    BS, S, H, D = cache_k.shape
    NN = k.shape[1]
    NR = S - NN - _SINK
    QUARTER = NR // 4

    lei = jnp.asarray(local_end_index, jnp.int32)
    nnt = jnp.asarray(num_new_tokens, jnp.int32)
    num_evicted = lei + nnt - S
    src0 = jnp.clip(_SINK + num_evicted, 0, S - NR).astype(jnp.int32)
    new_local_end = (lei + nnt - num_evicted).astype(jnp.int32)
    window_start = jnp.maximum(new_local_end - _MAX_ATTENTION_SIZE, 0).astype(jnp.int32)

    src0_v = jnp.full((16,), src0, jnp.int32)

    C = 40
    NBUF = 2
    NCH = QUARTER // C
    assert QUARTER % C == 0

    mesh = plsc.VectorSubcoreMesh(core_axis_name="c", subcore_axis_name="s")

    @functools.partial(
        pl.kernel,
        out_type=jax.ShapeDtypeStruct((BS, S, H, D), jnp.float32),
        mesh=mesh,
        scratch_types=[
            pltpu.VMEM((16,), jnp.int32),
        ] + [pltpu.SemaphoreType.DMA] * (2 * NBUF),
    )
    def _sc_copy(cache_h, new_h, s0_h, out_h, s0_vm, *sems):
        c = lax.axis_index("c")
        s = lax.axis_index("s")
        wid = s * 2 + c
        b = wid // 4
        q = wid % 4
        pltpu.sync_copy(s0_h, s0_vm)
        s0 = s0_vm[...][0]
        src_base = s0 + q * QUARTER
        dst_base = _SINK + q * QUARTER
        gsems = sems[:NBUF]
        ssems = sems[NBUF:]

        def scoped(*bufs):
            def gather(i):
                def mk(buf, sem):
                    return pltpu.make_async_copy(
                        cache_h.at[b, pl.ds(src_base + i * C, C)], buf, sem)
                return mk

            def scatter(i):
                def mk(buf, sem):
                    return pltpu.make_async_copy(
                        buf, out_h.at[b, pl.ds(dst_base + i * C, C)], sem)
                return mk

            _ring_copy([(gather(i), scatter(i)) for i in range(NCH)],
                       bufs, gsems, ssems, NBUF)

            def tail(tsrc_h, src_row, dst_row, nrows):
                off = 0
                while off < nrows:
                    m = min(C, nrows - off)
                    pltpu.sync_copy(tsrc_h.at[b, pl.ds(src_row + off, m)],
                                    bufs[0].at[pl.ds(0, m)])
                    pltpu.sync_copy(bufs[0].at[pl.ds(0, m)],
                                    out_h.at[b, pl.ds(dst_row + off, m)])
                    off += m

            @pl.when(q == 0)
            def _():
                tail(cache_h, 0, 0, _SINK)

            @pl.when(q == 3)
            def _():
                tail(new_h, 0, S - NN, NN)

        pl.run_scoped(scoped, *[pltpu.VMEM((C, H, D), jnp.float32)
                                for _ in range(NBUF)])

    ok = _sc_copy(cache_k, k, src0_v)
    ov = _sc_copy(cache_v, v, src0_v)

    return (ok, ov, window_start, new_local_end)
